# baseline (speedup 1.0000x reference)
import functools

import jax
import jax.numpy as jnp
from jax import lax
from jax.experimental import pallas as pl
from jax.experimental.pallas import tpu as pltpu
from jax.experimental.pallas import tpu_sc as plsc

_NC = 2
_NS = 16
_NW = _NC * _NS
_RPW = 8
_LANE = 128


def _gather_rows_sc(pe, idx, p, d):
    n_active = p // _RPW
    mesh = plsc.VectorSubcoreMesh(core_axis_name="c", subcore_axis_name="s")

    @functools.partial(
        pl.kernel,
        out_type=jax.ShapeDtypeStruct((p * d, _LANE), jnp.float32),
        mesh=mesh,
        compiler_params=pltpu.CompilerParams(use_tc_tiling_on_sc=False),
        scratch_types=[
            pltpu.VMEM((_RPW,), jnp.int32),
            pltpu.VMEM((_RPW, d, 1), jnp.float32),
            pltpu.SemaphoreType.DMA,
        ],
    )
    def gather_kernel(pe_hbm, idx_hbm, out_hbm, idx_v, rows_v, sem):
        wid = lax.axis_index("s") * _NC + lax.axis_index("c")
        base = wid * _RPW

        @pl.when(wid < n_active)
        def _():
            pltpu.sync_copy(idx_hbm.at[pl.ds(base, _RPW)], idx_v)
            pltpu.async_copy(pe_hbm.at[idx_v], rows_v, sem).wait()
            for k in range(_RPW):
                pltpu.sync_copy(
                    rows_v.at[k],
                    out_hbm.at[pl.ds((base + k) * d, d), pl.ds(0, 1)],
                )

    return gather_kernel(pe, idx)


def _add_tc_ring(x_t, pe_pad, rows_c, nbuf):
    m, n = x_t.shape
    nsteps = m // rows_c

    def body(x_hbm, pe_hbm, o_hbm, pe_v, ibufs, obufs, pe_sem, in_sems, out_sems):
        pltpu.make_async_copy(pe_hbm, pe_v, pe_sem).start()
        for s in range(nbuf):
            pltpu.make_async_copy(
                x_hbm.at[pl.ds(s * rows_c, rows_c)], ibufs.at[s], in_sems.at[s]
            ).start()
        pltpu.make_async_copy(pe_hbm, pe_v, pe_sem).wait()
        for i in range(nsteps):
            s = i % nbuf
            pltpu.make_async_copy(
                x_hbm.at[pl.ds(i * rows_c, rows_c)], ibufs.at[s], in_sems.at[s]
            ).wait()
            if i >= nbuf:
                pltpu.make_async_copy(
                    obufs.at[s], o_hbm.at[pl.ds((i - nbuf) * rows_c, rows_c)],
                    out_sems.at[s],
                ).wait()
            obufs[s] = ibufs[s] + pe_v[pl.ds(i * rows_c, rows_c), pl.ds(0, 1)]
            pltpu.make_async_copy(
                obufs.at[s], o_hbm.at[pl.ds(i * rows_c, rows_c)], out_sems.at[s]
            ).start()
            nxt = i + nbuf
            if nxt < nsteps:
                pltpu.make_async_copy(
                    x_hbm.at[pl.ds(nxt * rows_c, rows_c)], ibufs.at[s], in_sems.at[s]
                ).start()
        for i in range(max(0, nsteps - nbuf), nsteps):
            s = i % nbuf
            pltpu.make_async_copy(
                obufs.at[s], o_hbm.at[pl.ds(i * rows_c, rows_c)], out_sems.at[s]
            ).wait()

    return pl.pallas_call(
        body,
        in_specs=[
            pl.BlockSpec(memory_space=pltpu.HBM),
            pl.BlockSpec(memory_space=pltpu.HBM),
        ],
        out_specs=pl.BlockSpec(memory_space=pltpu.HBM),
        out_shape=jax.ShapeDtypeStruct((m, n), jnp.float32),
        scratch_shapes=[
            pltpu.VMEM((m, _LANE), jnp.float32),
            pltpu.VMEM((nbuf, rows_c, n), jnp.float32),
            pltpu.VMEM((nbuf, rows_c, n), jnp.float32),
            pltpu.SemaphoreType.DMA,
            pltpu.SemaphoreType.DMA((nbuf,)),
            pltpu.SemaphoreType.DMA((nbuf,)),
        ],
    )(x_t, pe_pad)


def kernel(input_data, index, pe):
    b, p, d = input_data.shape
    idx = index.astype(jnp.int32)
    pe_pad = _gather_rows_sc(pe[:, :, None], idx, p, d)
    x_t = input_data.transpose(1, 2, 0).reshape(p * d, b)
    out_t = _add_tc_ring(x_t, pe_pad, rows_c=200, nbuf=6)
    return out_t.reshape(p, d, b).transpose(2, 0, 1)
